# baseline (speedup 1.0000x reference)
import functools

import jax
import jax.numpy as jnp
from jax import lax
from jax.experimental import pallas as pl
from jax.experimental.pallas import tpu as pltpu
from jax.experimental.pallas import tpu_sc as plsc

_B = 4
_N = 4096
_S = 256
_T = _N - _S
_RT = 768
_NT = _T // _RT
_WPB = 4
_R0 = _S // _WPB
_R1 = _N // _WPB
_K0 = 4
_K1 = 8


def _sc_scan(rows_b, rows_f, cand_b, cand_f, out_ref, min_v,
             row0, nblk, nchunk, out0, kblk):
    _KBLK = kblk
    for kb in range(nblk):
        base = row0 + kb * 16 * _KBLK
        rxb = [rows_b[0, pl.ds(base + j * 16, 16)] for j in range(_KBLK)]
        ryb = [rows_b[1, pl.ds(base + j * 16, 16)] for j in range(_KBLK)]
        rzb = [rows_b[2, pl.ds(base + j * 16, 16)] for j in range(_KBLK)]
        rn = []
        for j in range(_KBLK):
            rfx = rows_f[0, pl.ds(base + j * 16, 16)]
            rfy = rows_f[1, pl.ds(base + j * 16, 16)]
            rfz = rows_f[2, pl.ds(base + j * 16, 16)]
            rn.append(rfx * rfx + rfy * rfy + rfz * rfz)
        inf = jnp.full((16,), jnp.float32(jnp.inf), jnp.float32)

        def mstep(mc, mins, rxb=rxb, ryb=ryb, rzb=rzb):
            qxv = cand_b[0, pl.ds(mc * 16, 16)]
            qyv = cand_b[1, pl.ds(mc * 16, 16)]
            qzv = cand_b[2, pl.ds(mc * 16, 16)]
            fxv = cand_f[0, pl.ds(mc * 16, 16)]
            fyv = cand_f[1, pl.ds(mc * 16, 16)]
            fzv = cand_f[2, pl.ds(mc * 16, 16)]
            qnv = fxv * fxv + fyv * fyv + fzv * fzv
            cur = list(mins)
            for i in range(16):
                qx = qxv[i]
                qy = qyv[i]
                qz = qzv[i]
                qn = qnv[i]
                for j in range(_KBLK):
                    t = rxb[j] * qx + ryb[j] * qy + rzb[j] * qz
                    dd = qn - 2.0 * t
                    cur[j] = jnp.minimum(cur[j], dd)
            return tuple(cur)

        mins = lax.fori_loop(0, nchunk, mstep, tuple(inf for _ in range(_KBLK)))
        for j in range(_KBLK):
            min_v[pl.ds(kb * 16 * _KBLK + j * 16, 16)] = jnp.maximum(
                rn[j] + mins[j], jnp.float32(0.0))
    pltpu.sync_copy(min_v.at[pl.ds(0, nblk * 16 * _KBLK)],
                    out_ref.at[pl.ds(out0, nblk * 16 * _KBLK)])


def _sc_body(stripb_hbm, stripf_hbm, fullb_hbm, fullf_hbm, out_hbm,
             sb_v, sf_v, fb_v, ff_v, min_v):
    d = lax.axis_index("c")
    s = lax.axis_index("s")
    b = s // _WPB
    q = s % _WPB

    pltpu.sync_copy(stripb_hbm.at[b], sb_v)
    pltpu.sync_copy(stripf_hbm.at[b], sf_v)
    pltpu.sync_copy(fullb_hbm.at[b], fb_v)
    pltpu.sync_copy(fullf_hbm.at[b], ff_v)

    @pl.when(d == 0)
    def _():
        _sc_scan(sb_v, sf_v, fb_v, ff_v, out_hbm.at[b], min_v,
                 q * _R0, _R0 // (16 * _K0), _N // 16, q * _R0, _K0)

    @pl.when(d == 1)
    def _():
        _sc_scan(fb_v, ff_v, sb_v, sf_v, out_hbm.at[b], min_v,
                 q * _R1, _R1 // (16 * _K1), _S // 16, _S + q * _R1, _K1)


_sc_minsq = functools.partial(
    pl.kernel,
    out_type=jax.ShapeDtypeStruct((_B, _S + _N), jnp.float32),
    mesh=plsc.VectorSubcoreMesh(core_axis_name="c", subcore_axis_name="s"),
    scratch_types=[
        pltpu.VMEM((3, _S), jnp.float32),
        pltpu.VMEM((3, _S), jnp.float32),
        pltpu.VMEM((3, _N), jnp.float32),
        pltpu.VMEM((3, _N), jnp.float32),
        pltpu.VMEM((_R1,), jnp.float32),
    ],
)(_sc_body)


def _tc_nn_body(lb_ref, lf_ref, rb_ref, rf_ref, rowd_ref, colmin_ref):
    t = pl.program_id(1)
    lb = lb_ref[0]
    rb = rb_ref[0]
    inner = lax.dot_general(lb, rb, (((1,), (0,)), ((), ())),
                            preferred_element_type=jnp.float32)
    lf = lf_ref[0]
    rf = rf_ref[0]
    n1 = jnp.sum(lf * lf, axis=1)
    n2 = jnp.sum(rf * rf, axis=0)
    rowmin = jnp.min(inner + n2[None, :], axis=1)
    rowd_ref[0, 0, :] = jnp.maximum(rowmin + n1, 0.0)
    part = jnp.min(inner + n1[:, None], axis=0)

    @pl.when(t == 0)
    def _():
        colmin_ref[0, 0, :] = part

    @pl.when(t != 0)
    def _():
        colmin_ref[0, 0, :] = jnp.minimum(colmin_ref[0, 0, :], part)


def _tc_minsq(lhs_b2, lhs_f, rhs_b, rhs_f):
    return pl.pallas_call(
        _tc_nn_body,
        grid=(_B, _NT),
        in_specs=[
            pl.BlockSpec((1, _RT, 8), lambda g, t: (g, t, 0)),
            pl.BlockSpec((1, _RT, 8), lambda g, t: (g, t, 0)),
            pl.BlockSpec((1, 8, _N), lambda g, t: (g, 0, 0)),
            pl.BlockSpec((1, 8, _N), lambda g, t: (g, 0, 0)),
        ],
        out_specs=[
            pl.BlockSpec((1, 1, _RT), lambda g, t: (g * _NT + t, 0, 0)),
            pl.BlockSpec((1, 1, _N), lambda g, t: (g, 0, 0)),
        ],
        out_shape=[
            jax.ShapeDtypeStruct((_B * _NT, 1, _RT), jnp.float32),
            jax.ShapeDtypeStruct((_B, 1, _N), jnp.float32),
        ],
    )(lhs_b2, lhs_f, rhs_b, rhs_f)


def _tc_reduce_body(d1_ref, cm_ref, scd2_ref, rf_ref, o_ref):
    n2 = jnp.sum(rf_ref[...] * rf_ref[...], axis=1)
    d2 = jnp.minimum(jnp.maximum(cm_ref[...] + n2, 0.0), scd2_ref[...])
    tot = jnp.sum(jnp.sqrt(d1_ref[...])) + jnp.sum(jnp.sqrt(d2))
    o_ref[0, 0] = tot * (1.0 / (2 * _B * _N))


def kernel(array1, array2):
    a1t = jnp.transpose(array1, (0, 2, 1))
    a2t = jnp.transpose(array2, (0, 2, 1))
    a1b16, a2b16 = lax.optimization_barrier(
        (a1t.astype(jnp.bfloat16), a2t.astype(jnp.bfloat16)))
    a1b = a1b16.astype(jnp.float32)
    a2b = a2b16.astype(jnp.float32)

    strip_b = a1b[..., _T:]
    strip_f = a1t[..., _T:]
    sc_out = _sc_minsq(strip_b, strip_f, a2b, a2t)
    scd1 = sc_out[:, :_S]
    scd2 = sc_out[:, _S:]

    pad1 = jnp.zeros((_B, 5, _N), jnp.float32)
    a1f8 = jnp.concatenate([a1t, pad1], axis=1)
    a1b8 = jnp.concatenate([a1b, pad1], axis=1)
    a2f8 = jnp.concatenate([a2t, pad1], axis=1)
    a2b8 = jnp.concatenate([a2b, pad1], axis=1)
    lhs_b2 = (-2.0 * a1b8).transpose(0, 2, 1)[:, :_T]
    lhs_f = a1f8.transpose(0, 2, 1)[:, :_T]
    rowd, colmin = _tc_minsq(lhs_b2, lhs_f, a2b8, a2f8)
    d1 = jnp.concatenate([rowd.reshape(_B, _T), scd1], axis=1)

    out = pl.pallas_call(
        _tc_reduce_body,
        out_shape=jax.ShapeDtypeStruct((1, 1), jnp.float32),
        out_specs=pl.BlockSpec(memory_space=pltpu.SMEM),
    )(d1, colmin.reshape(_B, _N), scd2, a2f8)
    return out[0, 0]


# speedup vs baseline: 1.9334x; 1.9334x over previous
import functools

import jax
import jax.numpy as jnp
from jax import lax
from jax.experimental import pallas as pl
from jax.experimental.pallas import tpu as pltpu
from jax.experimental.pallas import tpu_sc as plsc

_B = 4
_N = 4096
_S = 512
_T = _N - _S
_RT = 512
_NT = _T // _RT
_WPB = 4
_R0 = _S // _WPB
_R1 = _N // _WPB
_K0 = 8
_K1 = 8


def _sc_scan(rows_b, rows_f, cand_b, cand_f, out_ref, min_v,
             row0, nblk, nchunk, out0, kblk):
    _KBLK = kblk
    for kb in range(nblk):
        base = row0 + kb * 16 * _KBLK
        rxb = [rows_b[0, pl.ds(base + j * 16, 16)] for j in range(_KBLK)]
        ryb = [rows_b[1, pl.ds(base + j * 16, 16)] for j in range(_KBLK)]
        rzb = [rows_b[2, pl.ds(base + j * 16, 16)] for j in range(_KBLK)]
        rn = []
        for j in range(_KBLK):
            rfx = rows_f[0, pl.ds(base + j * 16, 16)]
            rfy = rows_f[1, pl.ds(base + j * 16, 16)]
            rfz = rows_f[2, pl.ds(base + j * 16, 16)]
            rn.append(rfx * rfx + rfy * rfy + rfz * rfz)
        inf = jnp.full((16,), jnp.float32(jnp.inf), jnp.float32)

        def mstep(mc, mins, rxb=rxb, ryb=ryb, rzb=rzb):
            qxv = cand_b[0, pl.ds(mc * 16, 16)]
            qyv = cand_b[1, pl.ds(mc * 16, 16)]
            qzv = cand_b[2, pl.ds(mc * 16, 16)]
            fxv = cand_f[0, pl.ds(mc * 16, 16)]
            fyv = cand_f[1, pl.ds(mc * 16, 16)]
            fzv = cand_f[2, pl.ds(mc * 16, 16)]
            qnv = fxv * fxv + fyv * fyv + fzv * fzv
            cur = list(mins)
            for i in range(16):
                qx = qxv[i]
                qy = qyv[i]
                qz = qzv[i]
                qn = qnv[i]
                for j in range(_KBLK):
                    t = rxb[j] * qx + ryb[j] * qy + rzb[j] * qz
                    dd = qn - 2.0 * t
                    cur[j] = jnp.minimum(cur[j], dd)
            return tuple(cur)

        mins = lax.fori_loop(0, nchunk, mstep, tuple(inf for _ in range(_KBLK)))
        for j in range(_KBLK):
            min_v[pl.ds(kb * 16 * _KBLK + j * 16, 16)] = jnp.maximum(
                rn[j] + mins[j], jnp.float32(0.0))
    pltpu.sync_copy(min_v.at[pl.ds(0, nblk * 16 * _KBLK)],
                    out_ref.at[pl.ds(out0, nblk * 16 * _KBLK)])


def _sc_body(stripb_hbm, stripf_hbm, fullb_hbm, fullf_hbm, out_hbm,
             sb_v, sf_v, fb_v, ff_v, min_v):
    d = lax.axis_index("c")
    s = lax.axis_index("s")
    b = s // _WPB
    q = s % _WPB

    pltpu.sync_copy(stripb_hbm.at[b], sb_v)
    pltpu.sync_copy(stripf_hbm.at[b], sf_v)
    pltpu.sync_copy(fullb_hbm.at[b], fb_v)
    pltpu.sync_copy(fullf_hbm.at[b], ff_v)

    @pl.when(d == 0)
    def _():
        _sc_scan(sb_v, sf_v, fb_v, ff_v, out_hbm.at[b], min_v,
                 q * _R0, _R0 // (16 * _K0), _N // 16, q * _R0, _K0)

    @pl.when(d == 1)
    def _():
        _sc_scan(fb_v, ff_v, sb_v, sf_v, out_hbm.at[b], min_v,
                 q * _R1, _R1 // (16 * _K1), _S // 16, _S + q * _R1, _K1)


_sc_minsq = functools.partial(
    pl.kernel,
    out_type=jax.ShapeDtypeStruct((_B, _S + _N), jnp.float32),
    mesh=plsc.VectorSubcoreMesh(core_axis_name="c", subcore_axis_name="s"),
    scratch_types=[
        pltpu.VMEM((3, _S), jnp.float32),
        pltpu.VMEM((3, _S), jnp.float32),
        pltpu.VMEM((3, _N), jnp.float32),
        pltpu.VMEM((3, _N), jnp.float32),
        pltpu.VMEM((_R1,), jnp.float32),
    ],
)(_sc_body)


def _tc_nn_body(lb_ref, lf_ref, rb_ref, rf_ref, rowd_ref, colmin_ref):
    t = pl.program_id(1)
    lb = lb_ref[0]
    rb = rb_ref[0]
    inner = lax.dot_general(lb, rb, (((1,), (0,)), ((), ())),
                            preferred_element_type=jnp.float32)
    lf = lf_ref[0]
    rf = rf_ref[0]
    n1 = jnp.sum(lf * lf, axis=1)
    n2 = jnp.sum(rf * rf, axis=0)
    rowmin = jnp.min(inner + n2[None, :], axis=1)
    rowd_ref[0, 0, :] = jnp.maximum(rowmin + n1, 0.0)
    part = jnp.min(inner + n1[:, None], axis=0)

    @pl.when(t == 0)
    def _():
        colmin_ref[0, 0, :] = part

    @pl.when(t != 0)
    def _():
        colmin_ref[0, 0, :] = jnp.minimum(colmin_ref[0, 0, :], part)


def _tc_minsq(lhs_b2, lhs_f, rhs_b, rhs_f):
    return pl.pallas_call(
        _tc_nn_body,
        grid=(_B, _NT),
        in_specs=[
            pl.BlockSpec((1, _RT, 8), lambda g, t: (g, t, 0)),
            pl.BlockSpec((1, _RT, 8), lambda g, t: (g, t, 0)),
            pl.BlockSpec((1, 8, _N), lambda g, t: (g, 0, 0)),
            pl.BlockSpec((1, 8, _N), lambda g, t: (g, 0, 0)),
        ],
        out_specs=[
            pl.BlockSpec((1, 1, _RT), lambda g, t: (g * _NT + t, 0, 0)),
            pl.BlockSpec((1, 1, _N), lambda g, t: (g, 0, 0)),
        ],
        out_shape=[
            jax.ShapeDtypeStruct((_B * _NT, 1, _RT), jnp.float32),
            jax.ShapeDtypeStruct((_B, 1, _N), jnp.float32),
        ],
    )(lhs_b2, lhs_f, rhs_b, rhs_f)


def _tc_reduce_body(d1_ref, cm_ref, scd2_ref, rf_ref, o_ref):
    n2 = jnp.sum(rf_ref[...] * rf_ref[...], axis=1)
    d2 = jnp.minimum(jnp.maximum(cm_ref[...] + n2, 0.0), scd2_ref[...])
    tot = jnp.sum(jnp.sqrt(d1_ref[...])) + jnp.sum(jnp.sqrt(d2))
    o_ref[0, 0] = tot * (1.0 / (2 * _B * _N))


def kernel(array1, array2):
    a1t = jnp.transpose(array1, (0, 2, 1))
    a2t = jnp.transpose(array2, (0, 2, 1))
    a1b16, a2b16 = lax.optimization_barrier(
        (a1t.astype(jnp.bfloat16), a2t.astype(jnp.bfloat16)))
    a1b = a1b16.astype(jnp.float32)
    a2b = a2b16.astype(jnp.float32)

    strip_b = a1b[..., _T:]
    strip_f = a1t[..., _T:]
    sc_out = _sc_minsq(strip_b, strip_f, a2b, a2t)
    scd1 = sc_out[:, :_S]
    scd2 = sc_out[:, _S:]

    pad1 = jnp.zeros((_B, 5, _N), jnp.float32)
    a1f8 = jnp.concatenate([a1t, pad1], axis=1)
    a1b8 = jnp.concatenate([a1b, pad1], axis=1)
    a2f8 = jnp.concatenate([a2t, pad1], axis=1)
    a2b8 = jnp.concatenate([a2b, pad1], axis=1)
    lhs_b2 = (-2.0 * a1b8).transpose(0, 2, 1)[:, :_T]
    lhs_f = a1f8.transpose(0, 2, 1)[:, :_T]
    rowd, colmin = _tc_minsq(lhs_b2, lhs_f, a2b8, a2f8)
    d1 = jnp.concatenate([rowd.reshape(_B, _T), scd1], axis=1)

    out = pl.pallas_call(
        _tc_reduce_body,
        out_shape=jax.ShapeDtypeStruct((1, 1), jnp.float32),
        out_specs=pl.BlockSpec(memory_space=pltpu.SMEM),
    )(d1, colmin.reshape(_B, _N), scd2, a2f8)
    return out[0, 0]
